# Initial kernel scaffold; baseline (speedup 1.0000x reference)
import jax
import jax.numpy as jnp
from jax.experimental import pallas as pl


def kernel(x_id, x_time, static_features, edges, edge_weight, dynamic_features, emb, W1, b1, W2, b2, Wg1, a_s1, a_d1, a_e1, We1, bg1, Wg2, a_s2, a_d2, a_e2, We2, bg2, W3, b3, W4, b4):
    raise NotImplementedError("write your pallas kernel here")


# speedup vs baseline: 5.4150x; 5.4150x over previous
import functools

import jax
import jax.numpy as jnp
from jax import lax
from jax.experimental import pallas as pl
from jax.experimental.pallas import tpu as pltpu
from jax.experimental.pallas import tpu_sc as plsc

N = 10000
E = 160000
D = 128
F = 8
V = 64
H = 2
T = 512
S = 6
B = 1024

NP = 10240
NB = 256
GRID = NP // NB

NC = 2
NS = 16
EPW = E // NS
CHUNK = 80
NCHUNK = EPW // CHUNK
GRP = CHUNK // 16
RPS = NP // NS


def _vgather_i(ref, idx):
    return plsc.load_gather(ref, [idx])


def _vgather(vec, idx):
    dnums = lax.GatherDimensionNumbers(
        offset_dims=(), collapsed_slice_dims=(0,), start_index_map=(0,))
    return lax.gather(vec, idx[:, None], dnums, (1,),
                      mode=lax.GatherScatterMode.PROMISE_IN_BOUNDS)


def _attn_tail(x1, Wg_ref, as_ref, ad_ref, h_ref, atab_ref):
    h0 = jnp.dot(x1, Wg_ref[:, :D], preferred_element_type=jnp.float32)
    h1 = jnp.dot(x1, Wg_ref[:, D:], preferred_element_type=jnp.float32)
    h_ref[0] = h0
    h_ref[1] = h1
    a0 = jnp.sum(h0 * as_ref[0:1, :], axis=1, keepdims=True)
    a1 = jnp.sum(h1 * as_ref[1:2, :], axis=1, keepdims=True)
    d0 = jnp.sum(h0 * ad_ref[0:1, :], axis=1, keepdims=True)
    d1 = jnp.sum(h1 * ad_ref[1:2, :], axis=1, keepdims=True)
    atab_ref[...] = jnp.concatenate(
        [a0, a1, d0, d1, jnp.zeros((NB, 12), jnp.float32)], axis=1)


def _ctab_write(ctab_ref, ae_ref, We_ref):
    ce0 = jnp.sum(We_ref[:, :D] * ae_ref[0:1, :])
    ce1 = jnp.sum(We_ref[:, D:] * ae_ref[1:2, :])
    col = lax.broadcasted_iota(jnp.int32, (8, 16), 1)
    ctab_ref[...] = jnp.where(col == 0, ce0, jnp.where(col == 1, ce1, 0.0))


def _tc1_body(sf_ref, emb_ref, W1_ref, b1_ref, W2_ref, b2_ref, Wg_ref,
              as_ref, ad_ref, ae_ref, We_ref,
              h_ref, atab_ref, ctab_ref, G_scr):
    @pl.when(pl.program_id(0) == 0)
    def _():
        for f in range(F):
            G_scr[pl.ds(f * V, V), :] = jnp.dot(
                emb_ref[f], W1_ref[pl.ds(f * D, D), :],
                preferred_element_type=jnp.float32)
        _ctab_write(ctab_ref, ae_ref, We_ref)

    sf = sf_ref[...]
    ohs = []
    for f in range(F):
        iot = lax.broadcasted_iota(jnp.int32, (NB, V), 1)
        ohs.append((sf[:, f:f + 1] == iot).astype(jnp.float32))
    oh = jnp.concatenate(ohs, axis=1)
    acc = b1_ref[...] + jnp.dot(oh, G_scr[...],
                                preferred_element_type=jnp.float32)
    x1 = jnp.dot(jax.nn.relu(acc), W2_ref[...],
                 preferred_element_type=jnp.float32) + b2_ref[...]
    _attn_tail(x1, Wg_ref, as_ref, ad_ref, h_ref, atab_ref)


def _tc2_body(num_ref, den_ref, bg_ref, Wg_ref, as_ref, ad_ref, ae_ref,
              We_ref, h_ref, atab_ref, ctab_ref):
    @pl.when(pl.program_id(0) == 0)
    def _():
        _ctab_write(ctab_ref, ae_ref, We_ref)

    g0 = num_ref[0] / (den_ref[0][:, None] + 1e-16)
    g1 = num_ref[1] / (den_ref[1][:, None] + 1e-16)
    x = jnp.concatenate([g0, g1], axis=1) + bg_ref[...]
    x = jnp.where(x > 0, x, jnp.exp(x) - 1.0)
    _attn_tail(x, Wg_ref, as_ref, ad_ref, h_ref, atab_ref)


def _tc3_body(num_ref, den_ref, bg_ref, W3_ref, b3_ref, W4_ref, b4_ref,
              out_ref):
    g0 = num_ref[0] / (den_ref[0][:, None] + 1e-16)
    g1 = num_ref[1] / (den_ref[1][:, None] + 1e-16)
    g = 0.5 * (g0 + g1) + bg_ref[...]
    z = jnp.dot(g, W3_ref[...], preferred_element_type=jnp.float32) \
        + b3_ref[...]
    out_ref[...] = jnp.dot(jax.nn.relu(z), W4_ref[...],
                           preferred_element_type=jnp.float32) + b4_ref[...]


def _sc_gat_body(src_h, dst_h, w_h, atabf_h, ctab_h, hflat_h, zrow_h, z1_h,
                 den_o, num_o,
                 den_sp, acc_sp, srcv, dstv, wv, hidxv, sidxv, didxv,
                 asv, adv, hrows, exv, ctv, semA, semB, semH):
    c = lax.axis_index("c")
    s = lax.axis_index("s")

    isl = pl.ds(s * RPS, RPS)
    pltpu.sync_copy(zrow_h.at[isl], acc_sp.at[isl])
    pltpu.sync_copy(z1_h.at[isl], den_sp.at[isl])
    pltpu.sync_copy(ctab_h.at[pl.ds(0, 16)], ctv)
    plsc.subcore_barrier()

    cvec = jnp.full((16,), 0, jnp.int32) + c
    cscal = _vgather(ctv[...], cvec)
    hoff = c * NP
    iot = lax.iota(jnp.int32, 16)

    def chunk_body(ch, carry):
        base = s * EPW + ch * CHUNK
        pltpu.sync_copy(src_h.at[pl.ds(base, CHUNK)], srcv)
        pltpu.sync_copy(dst_h.at[pl.ds(base, CHUNK)], dstv)
        pltpu.sync_copy(w_h.at[pl.ds(base, CHUNK)], wv)

        def idx_grp(g, carry2):
            sl = pl.ds(g * 16, 16)
            sv = srcv[sl]
            dv = dstv[sl]
            hidxv[sl] = sv + hoff
            sidxv[sl] = sv * 16 + c
            didxv[sl] = dv * 16 + (2 + c)
            return carry2
        lax.fori_loop(0, GRP, idx_grp, 0)
        cpA = pltpu.async_copy(atabf_h.at[sidxv], asv, semA)
        cpB = pltpu.async_copy(atabf_h.at[didxv], adv, semB)
        cpH = pltpu.async_copy(hflat_h.at[hidxv], hrows, semH)
        cpA.wait()
        cpB.wait()

        def ex_grp(g, carry2):
            sl = pl.ds(g * 16, 16)
            lg = asv[sl] + adv[sl] + wv[sl] * cscal
            lg = jnp.where(lg >= 0, lg, 0.2 * lg)
            exv[sl] = jnp.exp(lg)
            return carry2
        lax.fori_loop(0, GRP, ex_grp, 0)
        pltpu.sync_copy(exv, den_sp.at[dstv], add=True)
        cpH.wait()

        def sc_grp(g, carry2):
            exg = exv[pl.ds(g * 16, 16)]
            for j in range(16):
                row = jnp.full((16,), 0, jnp.int32) + (g * 16 + j)
                scal = _vgather(exg, jnp.full((16,), j, jnp.int32))
                for k in range(8):
                    colv = iot + (k * 16)
                    v = plsc.load_gather(hrows, [row, colv])
                    plsc.store_scatter(hrows, [row, colv], v * scal)
            return carry2
        lax.fori_loop(0, GRP, sc_grp, 0)
        pltpu.sync_copy(hrows, acc_sp.at[dstv], add=True)
        return carry

    lax.fori_loop(0, NCHUNK, chunk_body, 0)
    plsc.subcore_barrier()
    osl = pl.ds(c * NP + s * RPS, RPS)
    pltpu.sync_copy(den_sp.at[isl], den_o.at[osl])
    pltpu.sync_copy(acc_sp.at[isl], num_o.at[osl])


def _sc_dyn_body(xid_h, xt_h, dynf_h, out_h, xidv, xtv, idxv, vals, sem):
    c = lax.axis_index("c")
    s = lax.axis_index("s")
    wid = s * NC + c
    bpw = B // (NC * NS)
    base = wid * bpw
    pltpu.sync_copy(xid_h.at[pl.ds(base, bpw)], xidv)
    pltpu.sync_copy(xt_h.at[pl.ds(base, bpw)], xtv)
    iot = lax.iota(jnp.int32, 16)

    def idx_grp(g, carry):
        p = g * 16 + iot
        i = p // S
        k = p - i * S
        xi = _vgather_i(xidv, i)
        xt = _vgather_i(xtv, i)
        idxv[pl.ds(g * 16, 16)] = (xi * T + xt) * S + k
        return carry
    lax.fori_loop(0, (bpw * S) // 16, idx_grp, 0)
    half = (bpw * S) // 2
    pltpu.async_copy(dynf_h.at[idxv.at[pl.ds(0, half)]],
                     vals.at[pl.ds(0, half)], sem).wait()
    pltpu.async_copy(dynf_h.at[idxv.at[pl.ds(half, half)]],
                     vals.at[pl.ds(half, half)], sem).wait()
    pltpu.sync_copy(vals, out_h.at[pl.ds(base * S, bpw * S)])


def _make_mesh():
    return plsc.VectorSubcoreMesh(core_axis_name="c", subcore_axis_name="s",
                                  num_cores=NC, num_subcores=NS)


def _sc_gat(*args):
    return pl.kernel(
        _sc_gat_body, mesh=_make_mesh(),
        compiler_params=pltpu.CompilerParams(needs_layout_passes=False),
        out_type=(jax.ShapeDtypeStruct((NC * NP,), jnp.float32),
                  jax.ShapeDtypeStruct((NC * NP, D), jnp.float32)),
        scratch_types=[
            pltpu.VMEM_SHARED((NP,), jnp.float32),
            pltpu.VMEM_SHARED((NP, D), jnp.float32),
            pltpu.VMEM((CHUNK,), jnp.int32),
            pltpu.VMEM((CHUNK,), jnp.int32),
            pltpu.VMEM((CHUNK,), jnp.float32),
            pltpu.VMEM((CHUNK,), jnp.int32),
            pltpu.VMEM((CHUNK,), jnp.int32),
            pltpu.VMEM((CHUNK,), jnp.int32),
            pltpu.VMEM((CHUNK,), jnp.float32),
            pltpu.VMEM((CHUNK,), jnp.float32),
            pltpu.VMEM((CHUNK, D), jnp.float32),
            pltpu.VMEM((CHUNK,), jnp.float32),
            pltpu.VMEM((16,), jnp.float32),
            pltpu.SemaphoreType.DMA,
            pltpu.SemaphoreType.DMA,
            pltpu.SemaphoreType.DMA,
        ])(*args)


def _sc_dyn(*args):
    return pl.kernel(
        _sc_dyn_body, mesh=_make_mesh(),
        compiler_params=pltpu.CompilerParams(needs_layout_passes=False),
        out_type=jax.ShapeDtypeStruct((B * S,), jnp.float32),
        scratch_types=[
            pltpu.VMEM((B // (NC * NS),), jnp.int32),
            pltpu.VMEM((B // (NC * NS),), jnp.int32),
            pltpu.VMEM((B // (NC * NS) * S,), jnp.int32),
            pltpu.VMEM((B // (NC * NS) * S,), jnp.float32),
            pltpu.SemaphoreType.DMA,
        ])(*args)


def _tc1(sfp, emb, W1, b1, W2, b2, Wg1, a_s1, a_d1, a_e1, We1):
    return pl.pallas_call(
        _tc1_body,
        grid=(GRID,),
        in_specs=[
            pl.BlockSpec((NB, F), lambda i: (i, 0)),
            pl.BlockSpec((F, V, D), lambda i: (0, 0, 0)),
            pl.BlockSpec((F * D, D), lambda i: (0, 0)),
            pl.BlockSpec((1, D), lambda i: (0, 0)),
            pl.BlockSpec((D, D), lambda i: (0, 0)),
            pl.BlockSpec((1, D), lambda i: (0, 0)),
            pl.BlockSpec((D, H * D), lambda i: (0, 0)),
            pl.BlockSpec((H, D), lambda i: (0, 0)),
            pl.BlockSpec((H, D), lambda i: (0, 0)),
            pl.BlockSpec((H, D), lambda i: (0, 0)),
            pl.BlockSpec((1, H * D), lambda i: (0, 0)),
        ],
        out_specs=[
            pl.BlockSpec((H, NB, D), lambda i: (0, i, 0)),
            pl.BlockSpec((NB, 16), lambda i: (i, 0)),
            pl.BlockSpec((8, 16), lambda i: (0, 0)),
        ],
        out_shape=[
            jax.ShapeDtypeStruct((H, NP, D), jnp.float32),
            jax.ShapeDtypeStruct((NP, 16), jnp.float32),
            jax.ShapeDtypeStruct((8, 16), jnp.float32),
        ],
        scratch_shapes=[pltpu.VMEM((F * V, D), jnp.float32)],
    )(sfp, emb, W1, b1, W2, b2, Wg1, a_s1, a_d1, a_e1, We1)


def _tc2(num1, den1, bg1, Wg2, a_s2, a_d2, a_e2, We2):
    return pl.pallas_call(
        _tc2_body,
        grid=(GRID,),
        in_specs=[
            pl.BlockSpec((NC, NB, D), lambda i: (0, i, 0)),
            pl.BlockSpec((NC, NB), lambda i: (0, i)),
            pl.BlockSpec((1, H * D), lambda i: (0, 0)),
            pl.BlockSpec((H * D, H * D), lambda i: (0, 0)),
            pl.BlockSpec((H, D), lambda i: (0, 0)),
            pl.BlockSpec((H, D), lambda i: (0, 0)),
            pl.BlockSpec((H, D), lambda i: (0, 0)),
            pl.BlockSpec((1, H * D), lambda i: (0, 0)),
        ],
        out_specs=[
            pl.BlockSpec((H, NB, D), lambda i: (0, i, 0)),
            pl.BlockSpec((NB, 16), lambda i: (i, 0)),
            pl.BlockSpec((8, 16), lambda i: (0, 0)),
        ],
        out_shape=[
            jax.ShapeDtypeStruct((H, NP, D), jnp.float32),
            jax.ShapeDtypeStruct((NP, 16), jnp.float32),
            jax.ShapeDtypeStruct((8, 16), jnp.float32),
        ],
    )(num1, den1, bg1, Wg2, a_s2, a_d2, a_e2, We2)


def _tc3(num2, den2, bg2, W3, b3, W4, b4):
    return pl.pallas_call(
        _tc3_body,
        grid=(GRID,),
        in_specs=[
            pl.BlockSpec((NC, NB, D), lambda i: (0, i, 0)),
            pl.BlockSpec((NC, NB), lambda i: (0, i)),
            pl.BlockSpec((1, D), lambda i: (0, 0)),
            pl.BlockSpec((D, D), lambda i: (0, 0)),
            pl.BlockSpec((1, D), lambda i: (0, 0)),
            pl.BlockSpec((D, D), lambda i: (0, 0)),
            pl.BlockSpec((1, D), lambda i: (0, 0)),
        ],
        out_specs=pl.BlockSpec((NB, D), lambda i: (i, 0)),
        out_shape=jax.ShapeDtypeStruct((NP, D), jnp.float32),
    )(num2, den2, bg2, W3, b3, W4, b4)


def kernel(x_id, x_time, static_features, edges, edge_weight,
           dynamic_features, emb, W1, b1, W2, b2, Wg1, a_s1, a_d1, a_e1,
           We1, bg1, Wg2, a_s2, a_d2, a_e2, We2, bg2, W3, b3, W4, b4):
    sfp = jnp.pad(static_features, ((0, NP - N), (0, 0)))
    src = edges[0]
    dst = edges[1]
    zrow = jnp.zeros((NP, D), jnp.float32)
    z1 = jnp.zeros((NP,), jnp.float32)

    h1_t, atab1, ctab1 = _tc1(sfp, emb, W1, b1.reshape(1, D), W2,
                              b2.reshape(1, D), Wg1, a_s1, a_d1, a_e1, We1)
    den1, num1 = _sc_gat(src, dst, edge_weight, atab1.reshape(NP * 16),
                         ctab1.reshape(128), h1_t.reshape(H * NP, D),
                         zrow, z1)
    den1 = den1.reshape(NC, NP)
    num1 = num1.reshape(NC, NP, D)
    h2_t, atab2, ctab2 = _tc2(num1, den1, bg1.reshape(1, H * D), Wg2,
                              a_s2, a_d2, a_e2, We2)
    den2, num2 = _sc_gat(src, dst, edge_weight, atab2.reshape(NP * 16),
                         ctab2.reshape(128), h2_t.reshape(H * NP, D),
                         zrow, z1)
    den2 = den2.reshape(NC, NP)
    num2 = num2.reshape(NC, NP, D)
    se = _tc3(num2, den2, bg2.reshape(1, D), W3, b3.reshape(1, D), W4,
              b4.reshape(1, D))
    static_embedding = se[:N]

    dynf, _ = lax.optimization_barrier((dynamic_features, den2))
    dynamic_embedding = _sc_dyn(x_id, x_time, dynf.reshape(N * T * S))
    return (static_embedding, dynamic_embedding.reshape(B, S))


# speedup vs baseline: 9.0601x; 1.6731x over previous
import functools

import jax
import jax.numpy as jnp
from jax import lax
from jax.experimental import pallas as pl
from jax.experimental.pallas import tpu as pltpu
from jax.experimental.pallas import tpu_sc as plsc

N = 10000
E = 160000
D = 128
F = 8
V = 64
H = 2
T = 512
S = 6
B = 1024

NP = 10240
NB = 256
GRID = NP // NB

NC = 2
NS = 16
EPW = E // NS
CHUNK = 80
NCHUNK = EPW // CHUNK
GRP = CHUNK // 16
RPS = NP // NS
NBUF = 2


def _vgather_i(ref, idx):
    return plsc.load_gather(ref, [idx])


def _vgather(vec, idx):
    dnums = lax.GatherDimensionNumbers(
        offset_dims=(), collapsed_slice_dims=(0,), start_index_map=(0,))
    return lax.gather(vec, idx[:, None], dnums, (1,),
                      mode=lax.GatherScatterMode.PROMISE_IN_BOUNDS)


def _attn_tail(x1, Wg_ref, as_ref, ad_ref, h_ref, atab_ref):
    h0 = jnp.dot(x1, Wg_ref[:, :D], preferred_element_type=jnp.float32)
    h1 = jnp.dot(x1, Wg_ref[:, D:], preferred_element_type=jnp.float32)
    h_ref[0] = h0
    h_ref[1] = h1
    a0 = jnp.sum(h0 * as_ref[0:1, :], axis=1, keepdims=True)
    a1 = jnp.sum(h1 * as_ref[1:2, :], axis=1, keepdims=True)
    d0 = jnp.sum(h0 * ad_ref[0:1, :], axis=1, keepdims=True)
    d1 = jnp.sum(h1 * ad_ref[1:2, :], axis=1, keepdims=True)
    atab_ref[...] = jnp.concatenate(
        [a0, a1, d0, d1, jnp.zeros((NB, 12), jnp.float32)], axis=1)


def _ctab_write(ctab_ref, ae_ref, We_ref):
    ce0 = jnp.sum(We_ref[:, :D] * ae_ref[0:1, :])
    ce1 = jnp.sum(We_ref[:, D:] * ae_ref[1:2, :])
    col = lax.broadcasted_iota(jnp.int32, (8, 16), 1)
    ctab_ref[...] = jnp.where(col == 0, ce0, jnp.where(col == 1, ce1, 0.0))


def _tc1_body(sf_ref, emb_ref, W1_ref, b1_ref, W2_ref, b2_ref, Wg_ref,
              as_ref, ad_ref, ae_ref, We_ref,
              h_ref, atab_ref, ctab_ref, G_scr):
    @pl.when(pl.program_id(0) == 0)
    def _():
        for f in range(F):
            G_scr[pl.ds(f * V, V), :] = jnp.dot(
                emb_ref[f], W1_ref[pl.ds(f * D, D), :],
                preferred_element_type=jnp.float32)
        _ctab_write(ctab_ref, ae_ref, We_ref)

    sf = sf_ref[...]
    ohs = []
    for f in range(F):
        iot = lax.broadcasted_iota(jnp.int32, (NB, V), 1)
        ohs.append((sf[:, f:f + 1] == iot).astype(jnp.float32))
    oh = jnp.concatenate(ohs, axis=1)
    acc = b1_ref[...] + jnp.dot(oh, G_scr[...],
                                preferred_element_type=jnp.float32)
    x1 = jnp.dot(jax.nn.relu(acc), W2_ref[...],
                 preferred_element_type=jnp.float32) + b2_ref[...]
    _attn_tail(x1, Wg_ref, as_ref, ad_ref, h_ref, atab_ref)


def _tc2_body(num_ref, den_ref, bg_ref, Wg_ref, as_ref, ad_ref, ae_ref,
              We_ref, h_ref, atab_ref, ctab_ref):
    @pl.when(pl.program_id(0) == 0)
    def _():
        _ctab_write(ctab_ref, ae_ref, We_ref)

    g0 = num_ref[0] / (den_ref[0][:, None] + 1e-16)
    g1 = num_ref[1] / (den_ref[1][:, None] + 1e-16)
    x = jnp.concatenate([g0, g1], axis=1) + bg_ref[...]
    x = jnp.where(x > 0, x, jnp.exp(x) - 1.0)
    _attn_tail(x, Wg_ref, as_ref, ad_ref, h_ref, atab_ref)


def _tc3_body(num_ref, den_ref, bg_ref, W3_ref, b3_ref, W4_ref, b4_ref,
              out_ref):
    g0 = num_ref[0] / (den_ref[0][:, None] + 1e-16)
    g1 = num_ref[1] / (den_ref[1][:, None] + 1e-16)
    g = 0.5 * (g0 + g1) + bg_ref[...]
    z = jnp.dot(g, W3_ref[...], preferred_element_type=jnp.float32) \
        + b3_ref[...]
    out_ref[...] = jnp.dot(jax.nn.relu(z), W4_ref[...],
                           preferred_element_type=jnp.float32) + b4_ref[...]


def _sc_gat_body(src_h, dst_h, w_h, atabf_h, ctab_h, hflat_h, zrow_h, z1_h,
         den_o, num_o,
         den_sp, acc_sp, srcv, dstv, wv, hidxv, sidxv, didxv,
         asv, adv, hrows, exv, ctv,
         semL, semA, semB, semH, semD, semS):
    semAb = [semA.at[0], semA.at[1]]
    semBb = [semB.at[0], semB.at[1]]
    semHb = [semH.at[0], semH.at[1]]
    semDb = [semD.at[0], semD.at[1]]
    semSb = [semS.at[0], semS.at[1]]
    c = lax.axis_index("c")
    s = lax.axis_index("s")
    isl = pl.ds(s * RPS, RPS)
    pltpu.sync_copy(zrow_h.at[isl], acc_sp.at[isl])
    pltpu.sync_copy(z1_h.at[isl], den_sp.at[isl])
    pltpu.sync_copy(ctab_h.at[pl.ds(0, 16)], ctv)
    plsc.subcore_barrier()

    cvec = jnp.full((16,), 0, jnp.int32) + c
    cscal = _vgather(ctv[...], cvec)
    hoff = c * NP
    iot = lax.iota(jnp.int32, 16)

    def issue_front(ch, b):
        base = s * EPW + ch * CHUNK
        cpS = pltpu.async_copy(src_h.at[pl.ds(base, CHUNK)], srcv.at[b], semL)
        cpD = pltpu.async_copy(dst_h.at[pl.ds(base, CHUNK)], dstv.at[b], semL)
        cpW = pltpu.async_copy(w_h.at[pl.ds(base, CHUNK)], wv.at[b], semL)
        cpS.wait()
        cpD.wait()
        cpW.wait()

        def idx_grp(g, carry2):
            sl = pl.ds(g * 16, 16)
            sv = srcv.at[b][sl]
            dv = dstv.at[b][sl]
            hidxv.at[b][sl] = sv + hoff
            sidxv.at[b][sl] = sv * 16 + c
            didxv.at[b][sl] = dv * 16 + (2 + c)
            return carry2
        lax.fori_loop(0, GRP, idx_grp, 0)
        pltpu.async_copy(atabf_h.at[sidxv.at[b]], asv.at[b], semAb[b])
        pltpu.async_copy(atabf_h.at[didxv.at[b]], adv.at[b], semBb[b])
        pltpu.async_copy(hflat_h.at[hidxv.at[b]], hrows.at[b], semHb[b])

    def finish_back(b):
        pltpu.make_async_copy(atabf_h.at[sidxv.at[b]], asv.at[b], semAb[b]).wait()
        pltpu.make_async_copy(atabf_h.at[didxv.at[b]], adv.at[b], semBb[b]).wait()

        def ex_grp(g, carry2):
            sl = pl.ds(g * 16, 16)
            lg = asv.at[b][sl] + adv.at[b][sl] + wv.at[b][sl] * cscal
            lg = jnp.where(lg >= 0, lg, 0.2 * lg)
            exv.at[b][sl] = jnp.exp(lg)
            return carry2
        lax.fori_loop(0, GRP, ex_grp, 0)
        pltpu.async_copy(exv.at[b], den_sp.at[dstv.at[b]], semDb[b], add=True)
        pltpu.make_async_copy(hflat_h.at[hidxv.at[b]], hrows.at[b], semHb[b]).wait()

        def sc_grp(g, carry2):
            exg = exv.at[b][pl.ds(g * 16, 16)]
            for j in range(16):
                row = jnp.full((16,), 0, jnp.int32) + (g * 16 + j)
                scal = _vgather(exg, jnp.full((16,), j, jnp.int32))
                for k in range(8):
                    colv = iot + (k * 16)
                    v = plsc.load_gather(hrows.at[b], [row, colv])
                    plsc.store_scatter(hrows.at[b], [row, colv], v * scal)
            return carry2
        lax.fori_loop(0, GRP, sc_grp, 0)
        pltpu.async_copy(hrows.at[b], acc_sp.at[dstv.at[b]], semSb[b], add=True)

    def drain_scatters(b):
        pltpu.make_async_copy(exv.at[b], den_sp.at[dstv.at[b]], semDb[b]).wait()
        pltpu.make_async_copy(hrows.at[b], acc_sp.at[dstv.at[b]], semSb[b]).wait()

    issue_front(0, 0)

    def outer(i, carry):
        ch = 1 + i * NBUF

        def step(j):
            cc = ch + j
            b = (1 + j) % NBUF
            prev_b = j % NBUF

            @pl.when(cc < NCHUNK)
            def _():
                @pl.when(cc >= NBUF)
                def _():
                    drain_scatters(b)
                issue_front(cc, b)
                finish_back(prev_b)
        for j in range(NBUF):
            step(j)
        return carry
    lax.fori_loop(0, (NCHUNK + NBUF - 1) // NBUF, outer, 0)
    finish_back((NCHUNK - 1) % NBUF)
    drain_scatters(0)
    drain_scatters(1)

    plsc.subcore_barrier()
    osl = pl.ds(c * NP + s * RPS, RPS)
    pltpu.sync_copy(den_sp.at[isl], den_o.at[osl])
    pltpu.sync_copy(acc_sp.at[isl], num_o.at[osl])


def _sc_dyn_body(xid_h, xt_h, dynf_h, out_h, xidv, xtv, idxv, vals, sem):
    c = lax.axis_index("c")
    s = lax.axis_index("s")
    wid = s * NC + c
    bpw = B // (NC * NS)
    base = wid * bpw
    pltpu.sync_copy(xid_h.at[pl.ds(base, bpw)], xidv)
    pltpu.sync_copy(xt_h.at[pl.ds(base, bpw)], xtv)
    iot = lax.iota(jnp.int32, 16)

    def idx_grp(g, carry):
        p = g * 16 + iot
        i = p // S
        k = p - i * S
        xi = _vgather_i(xidv, i)
        xt = _vgather_i(xtv, i)
        idxv[pl.ds(g * 16, 16)] = (xi * T + xt) * S + k
        return carry
    lax.fori_loop(0, (bpw * S) // 16, idx_grp, 0)
    half = (bpw * S) // 2
    pltpu.async_copy(dynf_h.at[idxv.at[pl.ds(0, half)]],
                     vals.at[pl.ds(0, half)], sem).wait()
    pltpu.async_copy(dynf_h.at[idxv.at[pl.ds(half, half)]],
                     vals.at[pl.ds(half, half)], sem).wait()
    pltpu.sync_copy(vals, out_h.at[pl.ds(base * S, bpw * S)])


def _make_mesh():
    return plsc.VectorSubcoreMesh(core_axis_name="c", subcore_axis_name="s",
                                  num_cores=NC, num_subcores=NS)


def _sc_gat(*args):
    return pl.kernel(
        _sc_gat_body, mesh=_make_mesh(),
        compiler_params=pltpu.CompilerParams(needs_layout_passes=False),
        out_type=(jax.ShapeDtypeStruct((NC * NP,), jnp.float32),
                  jax.ShapeDtypeStruct((NC * NP, D), jnp.float32)),
        scratch_types=[
            pltpu.VMEM_SHARED((NP,), jnp.float32),
            pltpu.VMEM_SHARED((NP, D), jnp.float32),
            pltpu.VMEM((NBUF, CHUNK), jnp.int32),
            pltpu.VMEM((NBUF, CHUNK), jnp.int32),
            pltpu.VMEM((NBUF, CHUNK), jnp.float32),
            pltpu.VMEM((NBUF, CHUNK), jnp.int32),
            pltpu.VMEM((NBUF, CHUNK), jnp.int32),
            pltpu.VMEM((NBUF, CHUNK), jnp.int32),
            pltpu.VMEM((NBUF, CHUNK), jnp.float32),
            pltpu.VMEM((NBUF, CHUNK), jnp.float32),
            pltpu.VMEM((NBUF, CHUNK, D), jnp.float32),
            pltpu.VMEM((NBUF, CHUNK), jnp.float32),
            pltpu.VMEM((16,), jnp.float32),
            pltpu.SemaphoreType.DMA,
            pltpu.SemaphoreType.DMA((NBUF,)),
            pltpu.SemaphoreType.DMA((NBUF,)),
            pltpu.SemaphoreType.DMA((NBUF,)),
            pltpu.SemaphoreType.DMA((NBUF,)),
            pltpu.SemaphoreType.DMA((NBUF,)),
        ])(*args)


def _sc_dyn(*args):
    return pl.kernel(
        _sc_dyn_body, mesh=_make_mesh(),
        compiler_params=pltpu.CompilerParams(needs_layout_passes=False),
        out_type=jax.ShapeDtypeStruct((B * S,), jnp.float32),
        scratch_types=[
            pltpu.VMEM((B // (NC * NS),), jnp.int32),
            pltpu.VMEM((B // (NC * NS),), jnp.int32),
            pltpu.VMEM((B // (NC * NS) * S,), jnp.int32),
            pltpu.VMEM((B // (NC * NS) * S,), jnp.float32),
            pltpu.SemaphoreType.DMA,
        ])(*args)


def _tc1(sfp, emb, W1, b1, W2, b2, Wg1, a_s1, a_d1, a_e1, We1):
    return pl.pallas_call(
        _tc1_body,
        grid=(GRID,),
        in_specs=[
            pl.BlockSpec((NB, F), lambda i: (i, 0)),
            pl.BlockSpec((F, V, D), lambda i: (0, 0, 0)),
            pl.BlockSpec((F * D, D), lambda i: (0, 0)),
            pl.BlockSpec((1, D), lambda i: (0, 0)),
            pl.BlockSpec((D, D), lambda i: (0, 0)),
            pl.BlockSpec((1, D), lambda i: (0, 0)),
            pl.BlockSpec((D, H * D), lambda i: (0, 0)),
            pl.BlockSpec((H, D), lambda i: (0, 0)),
            pl.BlockSpec((H, D), lambda i: (0, 0)),
            pl.BlockSpec((H, D), lambda i: (0, 0)),
            pl.BlockSpec((1, H * D), lambda i: (0, 0)),
        ],
        out_specs=[
            pl.BlockSpec((H, NB, D), lambda i: (0, i, 0)),
            pl.BlockSpec((NB, 16), lambda i: (i, 0)),
            pl.BlockSpec((8, 16), lambda i: (0, 0)),
        ],
        out_shape=[
            jax.ShapeDtypeStruct((H, NP, D), jnp.float32),
            jax.ShapeDtypeStruct((NP, 16), jnp.float32),
            jax.ShapeDtypeStruct((8, 16), jnp.float32),
        ],
        scratch_shapes=[pltpu.VMEM((F * V, D), jnp.float32)],
    )(sfp, emb, W1, b1, W2, b2, Wg1, a_s1, a_d1, a_e1, We1)


def _tc2(num1, den1, bg1, Wg2, a_s2, a_d2, a_e2, We2):
    return pl.pallas_call(
        _tc2_body,
        grid=(GRID,),
        in_specs=[
            pl.BlockSpec((NC, NB, D), lambda i: (0, i, 0)),
            pl.BlockSpec((NC, NB), lambda i: (0, i)),
            pl.BlockSpec((1, H * D), lambda i: (0, 0)),
            pl.BlockSpec((H * D, H * D), lambda i: (0, 0)),
            pl.BlockSpec((H, D), lambda i: (0, 0)),
            pl.BlockSpec((H, D), lambda i: (0, 0)),
            pl.BlockSpec((H, D), lambda i: (0, 0)),
            pl.BlockSpec((1, H * D), lambda i: (0, 0)),
        ],
        out_specs=[
            pl.BlockSpec((H, NB, D), lambda i: (0, i, 0)),
            pl.BlockSpec((NB, 16), lambda i: (i, 0)),
            pl.BlockSpec((8, 16), lambda i: (0, 0)),
        ],
        out_shape=[
            jax.ShapeDtypeStruct((H, NP, D), jnp.float32),
            jax.ShapeDtypeStruct((NP, 16), jnp.float32),
            jax.ShapeDtypeStruct((8, 16), jnp.float32),
        ],
    )(num1, den1, bg1, Wg2, a_s2, a_d2, a_e2, We2)


def _tc3(num2, den2, bg2, W3, b3, W4, b4):
    return pl.pallas_call(
        _tc3_body,
        grid=(GRID,),
        in_specs=[
            pl.BlockSpec((NC, NB, D), lambda i: (0, i, 0)),
            pl.BlockSpec((NC, NB), lambda i: (0, i)),
            pl.BlockSpec((1, D), lambda i: (0, 0)),
            pl.BlockSpec((D, D), lambda i: (0, 0)),
            pl.BlockSpec((1, D), lambda i: (0, 0)),
            pl.BlockSpec((D, D), lambda i: (0, 0)),
            pl.BlockSpec((1, D), lambda i: (0, 0)),
        ],
        out_specs=pl.BlockSpec((NB, D), lambda i: (i, 0)),
        out_shape=jax.ShapeDtypeStruct((NP, D), jnp.float32),
    )(num2, den2, bg2, W3, b3, W4, b4)


def kernel(x_id, x_time, static_features, edges, edge_weight,
           dynamic_features, emb, W1, b1, W2, b2, Wg1, a_s1, a_d1, a_e1,
           We1, bg1, Wg2, a_s2, a_d2, a_e2, We2, bg2, W3, b3, W4, b4):
    sfp = jnp.pad(static_features, ((0, NP - N), (0, 0)))
    src = edges[0]
    dst = edges[1]
    zrow = jnp.zeros((NP, D), jnp.float32)
    z1 = jnp.zeros((NP,), jnp.float32)

    h1_t, atab1, ctab1 = _tc1(sfp, emb, W1, b1.reshape(1, D), W2,
                              b2.reshape(1, D), Wg1, a_s1, a_d1, a_e1, We1)
    den1, num1 = _sc_gat(src, dst, edge_weight, atab1.reshape(NP * 16),
                         ctab1.reshape(128), h1_t.reshape(H * NP, D),
                         zrow, z1)
    den1 = den1.reshape(NC, NP)
    num1 = num1.reshape(NC, NP, D)
    h2_t, atab2, ctab2 = _tc2(num1, den1, bg1.reshape(1, H * D), Wg2,
                              a_s2, a_d2, a_e2, We2)
    den2, num2 = _sc_gat(src, dst, edge_weight, atab2.reshape(NP * 16),
                         ctab2.reshape(128), h2_t.reshape(H * NP, D),
                         zrow, z1)
    den2 = den2.reshape(NC, NP)
    num2 = num2.reshape(NC, NP, D)
    se = _tc3(num2, den2, bg2.reshape(1, D), W3, b3.reshape(1, D), W4,
              b4.reshape(1, D))
    static_embedding = se[:N]

    dynf_flat = dynamic_features.reshape(N * T * S)
    dynf_flat, _ = lax.optimization_barrier((dynf_flat, den2))
    dynamic_embedding = _sc_dyn(x_id, x_time, dynf_flat)
    return (static_embedding, dynamic_embedding.reshape(B, S))
